# baseline (speedup 1.0000x reference)
import functools
import math

import jax
import jax.numpy as jnp
from jax import lax
from jax.experimental import pallas as pl
from jax.experimental.pallas import tpu as pltpu
from jax.experimental.pallas import tpu_sc as plsc

_B = 4096
_L = 200
_EMB = 64
_NC = 2
_NS = 16
_NW = _NC * _NS
_SEQ_PER_W = _B // _NW
_L0 = 128
_L1 = _L - _L0
_NLANE = 16
_NVEC = _EMB // _NLANE
_PAD = 128

_NBUF = 3
_TUNROLL = 8


def _gather_sum_body(tok_hbm, table_hbm, emb_hbm, sums_hbm,
                     tok_v, rows_v, sums_v,
                     gs0, gs1, gs2, ws0, ws1, ws2):
    gsems = (gs0, gs1, gs2)
    wsems = (ws0, ws1, ws2)
    wid = lax.axis_index("s") * _NC + lax.axis_index("c")
    seq0 = wid * _SEQ_PER_W
    pltpu.sync_copy(tok_hbm.at[pl.ds(seq0, _SEQ_PER_W)], tok_v)

    def issue_gather(b, k):
        pltpu.async_copy(
            table_hbm.at[tok_v.at[b, pl.ds(0, _L0)]],
            rows_v.at[k, pl.ds(0, _L0)], gsems[k])
        pltpu.async_copy(
            table_hbm.at[tok_v.at[b, pl.ds(_L0, _L1)]],
            rows_v.at[k, pl.ds(_L0, _L1)], gsems[k])

    def wait_bytes(k, sem):
        pltpu.make_async_copy(rows_v.at[k],
                              emb_hbm.at[0, pl.ds(0, _L), pl.ds(0, _EMB)],
                              sem).wait()

    wait_wbytes = wait_bytes

    for k in range(_NBUF - 1):
        issue_gather(k, k)

    def step(b, k):
        wait_bytes(k, gsems[k])
        pltpu.async_copy(
            rows_v.at[k],
            emb_hbm.at[seq0 + b, pl.ds(0, _L), pl.ds(0, _EMB)],
            wsems[k])
        def acc_body(tt, accs, k=k):
            new = list(accs)
            for i in range(_TUNROLL):
                t = tt * _TUNROLL + i
                for j in range(_NVEC):
                    new[j] = new[j] + rows_v[k, t, pl.ds(_NLANE * j,
                                                         _NLANE)]
            return tuple(new)
        z = jnp.zeros((_NLANE,), jnp.float32)
        accs = lax.fori_loop(0, _L // _TUNROLL, acc_body, (z,) * _NVEC)
        for j in range(_NVEC):
            sums_v[b, pl.ds(_NLANE * j, _NLANE)] = accs[j]
        kn = (k + _NBUF - 1) % _NBUF
        @pl.when(b == 0)
        def _():
            issue_gather(b + _NBUF - 1, kn)
        @pl.when((b >= 1) & (b <= _SEQ_PER_W - _NBUF))
        def _():
            wait_wbytes(kn, wsems[kn])
            issue_gather(b + _NBUF - 1, kn)

    _FULL = (_SEQ_PER_W // _NBUF) * _NBUF

    def group_body(g, carry):
        for k in range(_NBUF):
            step(g * _NBUF + k, k)
        return carry

    lax.fori_loop(0, _SEQ_PER_W // _NBUF, group_body, 0)
    for b in range(_FULL, _SEQ_PER_W):
        step(b, b % _NBUF)
    for k in range(_NBUF):
        wait_wbytes(k, wsems[k])
    pltpu.sync_copy(sums_v, sums_hbm.at[pl.ds(seq0, _SEQ_PER_W)])


_gather_sum = functools.partial(
    pl.kernel,
    mesh=plsc.VectorSubcoreMesh(core_axis_name="c", subcore_axis_name="s"),
    compiler_params=pltpu.CompilerParams(use_tc_tiling_on_sc=False),
    out_type=(
        jax.ShapeDtypeStruct((_B, _L, _PAD), jnp.float32),
        jax.ShapeDtypeStruct((_B, _EMB), jnp.float32),
    ),
    scratch_types=(
        pltpu.VMEM((_SEQ_PER_W, _L), jnp.int32),
        pltpu.VMEM((_NBUF, _L, _EMB), jnp.float32),
        pltpu.VMEM((_SEQ_PER_W, _EMB), jnp.float32),
        pltpu.SemaphoreType.DMA,
        pltpu.SemaphoreType.DMA,
        pltpu.SemaphoreType.DMA,
        pltpu.SemaphoreType.DMA,
        pltpu.SemaphoreType.DMA,
        pltpu.SemaphoreType.DMA,
    ),
)(_gather_sum_body)


def _transpose_body(src_hbm, out_hbm, buf_v, tbuf_v,
                    rs0, rs1, ws0t, ws1t):
    rsems = (rs0, rs1)
    wsems = (ws0t, ws1t)
    wid = lax.axis_index("s") * _NC + lax.axis_index("c")
    c = wid

    lane = lax.iota(jnp.int32, _NLANE)
    zero = jnp.zeros((_NLANE,), jnp.int32)
    jvecs = [lane + q * _NLANE for q in range(8)]

    def issue_read(g, k):
        pltpu.async_copy(
            src_hbm.at[pl.ds(c * 128, 128), pl.ds(g * 2, 2), pl.ds(0, _PAD)],
            buf_v.at[k], rsems[k])

    def wait_read(k):
        pltpu.make_async_copy(buf_v.at[k],
                              src_hbm.at[pl.ds(0, 128), pl.ds(0, 2),
                                         pl.ds(0, _PAD)],
                              rsems[k]).wait()

    def wait_write(u):
        pltpu.make_async_copy(tbuf_v.at[u],
                              out_hbm.at[0, :, pl.ds(0, 128)],
                              wsems[u]).wait()

    issue_read(0, 0)
    issue_read(1, 1)

    def g_body(gg, carry):
      for kk in range(2):
        g = gg * 2 + kk
        wait_read(kk)
        for u in range(2):
            l = g * 2 + u
            def o_body(o, carry2, kk=kk, u=u):
                for v in range(8):
                    evec = (lane + o * 8 + v) & (_EMB - 1)
                    for q in range(8):
                        vals = plsc.load_gather(buf_v.at[kk],
                                                [jvecs[q], zero + u, evec])
                        plsc.store_scatter(tbuf_v.at[u], [evec, jvecs[q]],
                                           vals)
                return carry2
            lax.fori_loop(0, _EMB // 8, o_body, 0)
            @pl.when(l >= 2)
            def _():
                wait_write(u)
            pltpu.async_copy(tbuf_v.at[u],
                             out_hbm.at[l, :, pl.ds(c * 128, 128)],
                             wsems[u])
        @pl.when(g <= _L // 2 - 3)
        def _():
            issue_read(g + 2, kk)
      return carry

    lax.fori_loop(0, _L // 4, g_body, 0)
    wait_write(0)
    wait_write(1)


_transpose = functools.partial(
    pl.kernel,
    mesh=plsc.VectorSubcoreMesh(core_axis_name="c", subcore_axis_name="s"),
    compiler_params=pltpu.CompilerParams(use_tc_tiling_on_sc=True,
                                        needs_layout_passes=False),
    out_type=(jax.ShapeDtypeStruct((_L, _EMB, _B), jnp.float32),),
    scratch_types=(
        pltpu.VMEM((2, 128, 2, _PAD), jnp.float32),
        pltpu.VMEM((2, _EMB, 128), jnp.float32),
        pltpu.SemaphoreType.DMA,
        pltpu.SemaphoreType.DMA,
        pltpu.SemaphoreType.DMA,
        pltpu.SemaphoreType.DMA,
    ),
)(_transpose_body)


def _gelu(x):
    return 0.5 * x * (1.0 + lax.erf(x * (1.0 / math.sqrt(2.0))))


def _mlp_body(s_ref, w1_ref, b1_ref, w2_ref, b2_ref, w3_ref, b3_ref, o_ref):
    x = s_ref[...] * (1.0 / _L)
    h = lax.dot_general(x, w1_ref[...], (((1,), (1,)), ((), ())),
                        preferred_element_type=jnp.float32)
    h = _gelu(h + b1_ref[...])
    h = lax.dot_general(h, w2_ref[...], (((1,), (1,)), ((), ())),
                        preferred_element_type=jnp.float32)
    h = _gelu(h + b2_ref[...])
    e = lax.dot_general(w3_ref[...], h, (((1,), (1,)), ((), ())),
                        preferred_element_type=jnp.float32)
    o_ref[...] = e + b3_ref[0]


def kernel(token_ids, table, W1, b1, W2, b2, W3, b3):
    emb_pad, sums = _gather_sum(token_ids, table)
    (emb_t,) = _transpose(emb_pad)
    emb = emb_t.transpose(2, 0, 1)
    energy_row = pl.pallas_call(
        _mlp_body,
        in_specs=[
            pl.BlockSpec(memory_space=pltpu.VMEM),
            pl.BlockSpec(memory_space=pltpu.VMEM),
            pl.BlockSpec(memory_space=pltpu.VMEM),
            pl.BlockSpec(memory_space=pltpu.VMEM),
            pl.BlockSpec(memory_space=pltpu.VMEM),
            pl.BlockSpec(memory_space=pltpu.VMEM),
            pl.BlockSpec(memory_space=pltpu.SMEM),
        ],
        out_specs=pl.BlockSpec(memory_space=pltpu.VMEM),
        out_shape=jax.ShapeDtypeStruct((1, _B), jnp.float32),
    )(sums, W1, b1.reshape(1, -1), W2, b2.reshape(1, -1), W3, b3)
    return energy_row.reshape(_B, 1), emb


# speedup vs baseline: 1.0294x; 1.0294x over previous
import functools
import math

import jax
import jax.numpy as jnp
from jax import lax
from jax.experimental import pallas as pl
from jax.experimental.pallas import tpu as pltpu
from jax.experimental.pallas import tpu_sc as plsc

_B = 4096
_L = 200
_EMB = 64
_NC = 2
_NS = 16
_NW = _NC * _NS
_SEQ_PER_W = _B // _NW
_L0 = 128
_L1 = _L - _L0
_NLANE = 16
_NVEC = _EMB // _NLANE
_PAD = 128

_NBUF = 3
_TUNROLL = 8


def _gather_sum_body(tok_hbm, table_hbm, emb_hbm, sums_hbm,
                     tok_v, rows_v, sums_v,
                     gs0, gs1, gs2, ws0, ws1, ws2):
    gsems = (gs0, gs1, gs2)
    wsems = (ws0, ws1, ws2)
    wid = lax.axis_index("s") * _NC + lax.axis_index("c")
    seq0 = wid * _SEQ_PER_W
    pltpu.sync_copy(tok_hbm.at[pl.ds(seq0, _SEQ_PER_W)], tok_v)

    def issue_gather(b, k):
        pltpu.async_copy(
            table_hbm.at[tok_v.at[b, pl.ds(0, _L0)]],
            rows_v.at[k, pl.ds(0, _L0)], gsems[k])
        pltpu.async_copy(
            table_hbm.at[tok_v.at[b, pl.ds(_L0, _L1)]],
            rows_v.at[k, pl.ds(_L0, _L1)], gsems[k])

    def wait_bytes(k, sem):
        pltpu.make_async_copy(rows_v.at[k],
                              emb_hbm.at[0, pl.ds(0, _L), pl.ds(0, _EMB)],
                              sem).wait()

    wait_wbytes = wait_bytes

    for k in range(_NBUF - 1):
        issue_gather(k, k)

    def step(b, k):
        wait_bytes(k, gsems[k])
        pltpu.async_copy(
            rows_v.at[k],
            emb_hbm.at[seq0 + b, pl.ds(0, _L), pl.ds(0, _EMB)],
            wsems[k])
        def acc_body(tt, accs, k=k):
            new = list(accs)
            for i in range(_TUNROLL):
                t = tt * _TUNROLL + i
                for j in range(_NVEC):
                    new[j] = new[j] + rows_v[k, t, pl.ds(_NLANE * j,
                                                         _NLANE)]
            return tuple(new)
        z = jnp.zeros((_NLANE,), jnp.float32)
        accs = lax.fori_loop(0, _L // _TUNROLL, acc_body, (z,) * _NVEC)
        for j in range(_NVEC):
            sums_v[b, pl.ds(_NLANE * j, _NLANE)] = accs[j]
        kn = (k + _NBUF - 1) % _NBUF
        @pl.when(b == 0)
        def _():
            issue_gather(b + _NBUF - 1, kn)
        @pl.when((b >= 1) & (b <= _SEQ_PER_W - _NBUF))
        def _():
            wait_wbytes(kn, wsems[kn])
            issue_gather(b + _NBUF - 1, kn)

    _FULL = (_SEQ_PER_W // _NBUF) * _NBUF

    def group_body(g, carry):
        for k in range(_NBUF):
            step(g * _NBUF + k, k)
        return carry

    lax.fori_loop(0, _SEQ_PER_W // _NBUF, group_body, 0)
    for b in range(_FULL, _SEQ_PER_W):
        step(b, b % _NBUF)
    for k in range(_NBUF):
        wait_wbytes(k, wsems[k])
    pltpu.sync_copy(sums_v, sums_hbm.at[pl.ds(seq0, _SEQ_PER_W)])


_gather_sum = functools.partial(
    pl.kernel,
    mesh=plsc.VectorSubcoreMesh(core_axis_name="c", subcore_axis_name="s"),
    compiler_params=pltpu.CompilerParams(use_tc_tiling_on_sc=False),
    out_type=(
        jax.ShapeDtypeStruct((_B, _L, _PAD), jnp.float32),
        jax.ShapeDtypeStruct((_B, _EMB), jnp.float32),
    ),
    scratch_types=(
        pltpu.VMEM((_SEQ_PER_W, _L), jnp.int32),
        pltpu.VMEM((_NBUF, _L, _EMB), jnp.float32),
        pltpu.VMEM((_SEQ_PER_W, _EMB), jnp.float32),
        pltpu.SemaphoreType.DMA,
        pltpu.SemaphoreType.DMA,
        pltpu.SemaphoreType.DMA,
        pltpu.SemaphoreType.DMA,
        pltpu.SemaphoreType.DMA,
        pltpu.SemaphoreType.DMA,
    ),
)(_gather_sum_body)


def _transpose_body(src_hbm, out_hbm, buf_v, tbuf_v,
                    rs0, rs1, ws0t, ws1t):
    rsems = (rs0, rs1)
    wsems = (ws0t, ws1t)
    wid = lax.axis_index("s") * _NC + lax.axis_index("c")
    c = wid

    lane = lax.iota(jnp.int32, _NLANE)
    zero = jnp.zeros((_NLANE,), jnp.int32)
    jvecs = [lane + q * _NLANE for q in range(8)]

    def issue_read(l, k):
        pltpu.async_copy(
            src_hbm.at[pl.ds(c * 128, 128), pl.ds(l, 1), pl.ds(0, _PAD)],
            buf_v.at[k], rsems[k])

    def wait_read(k):
        pltpu.make_async_copy(buf_v.at[k],
                              src_hbm.at[pl.ds(0, 128), pl.ds(0, 1),
                                         pl.ds(0, _PAD)],
                              rsems[k]).wait()

    def wait_write(k):
        pltpu.make_async_copy(tbuf_v.at[k],
                              out_hbm.at[0, :, pl.ds(0, 128)],
                              wsems[k]).wait()

    issue_read(0, 0)
    issue_read(1, 1)

    def l_body(g, carry):
      for k in range(2):
        l = g * 2 + k
        wait_read(k)
        def o_body(o, carry2, k=k):
            for u in range(8):
                evec = (lane + o * 8 + u) & (_EMB - 1)
                for q in range(8):
                    vals = plsc.load_gather(buf_v.at[k],
                                            [jvecs[q], zero, evec])
                    plsc.store_scatter(tbuf_v.at[k], [evec, jvecs[q]], vals)
            return carry2
        lax.fori_loop(0, _EMB // 8, o_body, 0)
        @pl.when(l >= 2)
        def _():
            wait_write(k)
        pltpu.async_copy(tbuf_v.at[k],
                         out_hbm.at[l, :, pl.ds(c * 128, 128)],
                         wsems[k])
        @pl.when(l <= _L - 3)
        def _():
            issue_read(l + 2, k)
      return carry

    lax.fori_loop(0, _L // 2, l_body, 0)
    wait_write(0)
    wait_write(1)


_transpose = functools.partial(
    pl.kernel,
    mesh=plsc.VectorSubcoreMesh(core_axis_name="c", subcore_axis_name="s"),
    compiler_params=pltpu.CompilerParams(use_tc_tiling_on_sc=True,
                                        needs_layout_passes=False),
    out_type=(jax.ShapeDtypeStruct((_L, _EMB, _B), jnp.float32),),
    scratch_types=(
        pltpu.VMEM((2, 128, 1, _PAD), jnp.float32),
        pltpu.VMEM((2, _EMB, 128), jnp.float32),
        pltpu.SemaphoreType.DMA,
        pltpu.SemaphoreType.DMA,
        pltpu.SemaphoreType.DMA,
        pltpu.SemaphoreType.DMA,
    ),
)(_transpose_body)


def _gelu(x):
    return 0.5 * x * (1.0 + lax.erf(x * (1.0 / math.sqrt(2.0))))


def _mlp_body(s_ref, w1_ref, b1_ref, w2_ref, b2_ref, w3_ref, b3_ref, o_ref):
    x = s_ref[...] * (1.0 / _L)
    h = lax.dot_general(x, w1_ref[...], (((1,), (1,)), ((), ())),
                        preferred_element_type=jnp.float32)
    h = _gelu(h + b1_ref[...])
    h = lax.dot_general(h, w2_ref[...], (((1,), (1,)), ((), ())),
                        preferred_element_type=jnp.float32)
    h = _gelu(h + b2_ref[...])
    e = lax.dot_general(w3_ref[...], h, (((1,), (1,)), ((), ())),
                        preferred_element_type=jnp.float32)
    o_ref[...] = e + b3_ref[0]


def kernel(token_ids, table, W1, b1, W2, b2, W3, b3):
    emb_pad, sums = _gather_sum(token_ids, table)
    (emb_t,) = _transpose(emb_pad)
    emb = emb_t.transpose(2, 0, 1)
    energy_row = pl.pallas_call(
        _mlp_body,
        in_specs=[
            pl.BlockSpec(memory_space=pltpu.VMEM),
            pl.BlockSpec(memory_space=pltpu.VMEM),
            pl.BlockSpec(memory_space=pltpu.VMEM),
            pl.BlockSpec(memory_space=pltpu.VMEM),
            pl.BlockSpec(memory_space=pltpu.VMEM),
            pl.BlockSpec(memory_space=pltpu.VMEM),
            pl.BlockSpec(memory_space=pltpu.SMEM),
        ],
        out_specs=pl.BlockSpec(memory_space=pltpu.VMEM),
        out_shape=jax.ShapeDtypeStruct((1, _B), jnp.float32),
    )(sums, W1, b1.reshape(1, -1), W2, b2.reshape(1, -1), W3, b3)
    return energy_row.reshape(_B, 1), emb
